# baseline (speedup 1.0000x reference)
import jax
import jax.numpy as jnp
from jax.experimental import pallas as pl
from jax.experimental.pallas import tpu as pltpu

_CHUNK_ROWS = (512, 512, 512, 512, 512, 512, 512, 512)


def _make_body(batch, pos, d):
    starts = []
    off = 0
    for r in _CHUNK_ROWS:
        starts.append(off)
        off += r
    assert off == pos

    def body(w_hbm, o_hbm, vmem, sem_in, sem_out):
        ins = []
        for i, (s, r) in enumerate(zip(starts, _CHUNK_ROWS)):
            c = pltpu.make_async_copy(
                w_hbm.at[pl.ds(s, r), :],
                vmem.at[pl.ds(s, r), :],
                sem_in.at[i],
            )
            c.start()
            ins.append(c)
        for c in ins:
            c.wait()
        outs = []
        for i, (s, r) in enumerate(zip(starts, _CHUNK_ROWS)):
            for b in range(batch):
                c = pltpu.make_async_copy(
                    vmem.at[pl.ds(s, r), :],
                    o_hbm.at[b, pl.ds(s, r), :],
                    sem_out.at[i, b],
                )
                c.start()
                outs.append(c)
        for c in outs:
            c.wait()

    return body


def kernel(tokens, W_pos):
    batch, pos = tokens.shape
    n_ctx, d = W_pos.shape
    n = len(_CHUNK_ROWS)
    out = pl.pallas_call(
        _make_body(batch, pos, d),
        in_specs=[pl.BlockSpec(memory_space=pl.ANY)],
        out_specs=pl.BlockSpec(memory_space=pl.ANY),
        out_shape=jax.ShapeDtypeStruct((batch, pos, d), W_pos.dtype),
        scratch_shapes=[
            pltpu.VMEM((pos, d), W_pos.dtype),
            pltpu.SemaphoreType.DMA((n,)),
            pltpu.SemaphoreType.DMA((n, 2)),
        ],
    )(W_pos)
    return out


# speedup vs baseline: 1.1109x; 1.1109x over previous
import jax
import jax.numpy as jnp
from jax.experimental import pallas as pl
from jax.experimental.pallas import tpu as pltpu

_CHUNK_ROWS = (128, 256, 512, 1024, 1024, 1024, 128)


def _make_body(batch, pos, d):
    starts = []
    off = 0
    for r in _CHUNK_ROWS:
        starts.append(off)
        off += r
    assert off == pos

    def body(w_hbm, o_hbm, vmem, sem_in, sem_out):
        ins = []
        for i, (s, r) in enumerate(zip(starts, _CHUNK_ROWS)):
            c = pltpu.make_async_copy(
                w_hbm.at[pl.ds(s, r), :],
                vmem.at[pl.ds(s, r), :],
                sem_in.at[i],
            )
            c.start()
            ins.append(c)
        outs = []
        for i, (s, r) in enumerate(zip(starts, _CHUNK_ROWS)):
            ins[i].wait()
            for b in range(batch):
                c = pltpu.make_async_copy(
                    vmem.at[pl.ds(s, r), :],
                    o_hbm.at[b, pl.ds(s, r), :],
                    sem_out.at[i, b],
                )
                c.start()
                outs.append(c)
        for c in outs:
            c.wait()

    return body


def kernel(tokens, W_pos):
    batch, pos = tokens.shape
    n_ctx, d = W_pos.shape
    n = len(_CHUNK_ROWS)
    out = pl.pallas_call(
        _make_body(batch, pos, d),
        in_specs=[pl.BlockSpec(memory_space=pl.ANY)],
        out_specs=pl.BlockSpec(memory_space=pl.ANY),
        out_shape=jax.ShapeDtypeStruct((batch, pos, d), W_pos.dtype),
        scratch_shapes=[
            pltpu.VMEM((pos, d), W_pos.dtype),
            pltpu.SemaphoreType.DMA((n,)),
            pltpu.SemaphoreType.DMA((n, 2)),
        ],
    )(W_pos)
    return out
